# baseline (speedup 1.0000x reference)
import functools

import jax
import jax.numpy as jnp
from jax import lax
from jax.experimental import pallas as pl
from jax.experimental.pallas import tpu as pltpu
from jax.experimental.pallas import tpu_sc as plsc


def _softmax_pad_body(x_ref, o_ref):
    x = x_ref[...]
    m = jnp.max(x, axis=-1, keepdims=True)
    e = jnp.exp(x - m)
    s = jnp.sum(e, axis=-1, keepdims=True)
    d = x.shape[-1]
    if o_ref.shape[-1] == d:
        o_ref[...] = e / s
    else:
        o_ref[:, :d] = e / s
        o_ref[:, d:] = jnp.zeros(
            (x.shape[0], o_ref.shape[-1] - d), jnp.float32)


def _make_sc_gather(B, D, DP, NC, NS, CH):
    NW = NC * NS
    b_per_w = B // NW
    n_ch = b_per_w // CH
    mesh = plsc.VectorSubcoreMesh(core_axis_name="c", subcore_axis_name="s")

    @functools.partial(
        pl.kernel,
        mesh=mesh,
        out_type=jax.ShapeDtypeStruct((B, D), jnp.float32),
        compiler_params=pltpu.CompilerParams(use_tc_tiling_on_sc=False),
        scratch_types=[
            pltpu.VMEM((b_per_w,), jnp.int32),
            pltpu.VMEM((CH, DP), jnp.float32),
            pltpu.VMEM((CH, DP), jnp.float32),
            pltpu.SemaphoreType.DMA,
            pltpu.SemaphoreType.DMA,
            pltpu.SemaphoreType.DMA,
            pltpu.SemaphoreType.DMA,
        ],
    )
    def gather_rows(table_hbm, idx_hbm, out_hbm, idx_v, buf0, buf1,
                    gsem0, gsem1, ssem0, ssem1):
        wid = lax.axis_index("s") * NC + lax.axis_index("c")
        base = wid * b_per_w
        pltpu.sync_copy(idx_hbm.at[pl.ds(base, b_per_w)], idx_v)

        bufs = (buf0, buf1)
        gsems = (gsem0, gsem1)
        ssems = (ssem0, ssem1)

        def gather(c):
            return pltpu.async_copy(
                table_hbm.at[idx_v.at[pl.ds(c * CH, CH)]],
                bufs[c % 2], gsems[c % 2])

        def scatter(c):
            src = bufs[c % 2]
            if DP != D:
                src = src.at[:, pl.ds(0, D)]
            return pltpu.async_copy(
                src, out_hbm.at[pl.ds(base + c * CH, CH)],
                ssems[c % 2])

        gathers = {0: gather(0)}
        scatters = {}
        for c in range(n_ch):
            if c + 1 < n_ch:
                if c - 1 >= 0:
                    scatters[c - 1].wait()
                gathers[c + 1] = gather(c + 1)
            gathers[c].wait()
            scatters[c] = scatter(c)
        if n_ch >= 2:
            scatters[n_ch - 2].wait()
        scatters[n_ch - 1].wait()

    return gather_rows


def kernel(inputs_idx, params):
    B = inputs_idx.shape[0]
    N, D = params.shape
    DP = D

    sm_table = pl.pallas_call(
        _softmax_pad_body,
        out_shape=jax.ShapeDtypeStruct((N, DP), jnp.float32),
    )(params)

    info = plsc.get_sparse_core_info()
    NC, NS = info.num_cores, info.num_subcores
    gather_rows = _make_sc_gather(B, D, DP, NC, NS, CH=32)
    return gather_rows(sm_table, inputs_idx.astype(jnp.int32))


# speedup vs baseline: 1.3130x; 1.3130x over previous
import functools

import jax
import jax.numpy as jnp
from jax import lax
from jax.experimental import pallas as pl
from jax.experimental.pallas import tpu as pltpu
from jax.experimental.pallas import tpu_sc as plsc


def _softmax_pad_body(x_ref, o_ref):
    x = x_ref[...]
    m = jnp.max(x, axis=-1, keepdims=True)
    e = jnp.exp(x - m)
    s = jnp.sum(e, axis=-1, keepdims=True)
    d = x.shape[-1]
    if o_ref.shape[-1] == d:
        o_ref[...] = e / s
    else:
        o_ref[:, :d] = e / s
        o_ref[:, d:] = jnp.zeros(
            (x.shape[0], o_ref.shape[-1] - d), jnp.float32)


def _tail_body(idx_ref, table_ref, o_ref, *, n, c0, c1):
    idx = idx_ref[0, 0, :]
    onehot = (idx[:, None] == lax.broadcasted_iota(jnp.int32, (1, n), 1))
    tail = table_ref[:, c0:c1]
    o_ref[...] = jnp.dot(onehot.astype(jnp.float32), tail,
                         preferred_element_type=jnp.float32)


def _make_sc_gather(B, D, DP, DA, NC, NS, CH):
    DT = D - DA
    NW = NC * NS
    b_per_w = B // NW
    n_ch = b_per_w // CH
    mesh = plsc.VectorSubcoreMesh(core_axis_name="c", subcore_axis_name="s")

    scratch = [
        pltpu.VMEM((b_per_w,), jnp.int32),
        pltpu.VMEM((CH, DP), jnp.float32),
        pltpu.VMEM((CH, DP), jnp.float32),
    ]
    if DT:
        scratch += [
            pltpu.VMEM((CH, DT), jnp.float32),
            pltpu.VMEM((CH, DT), jnp.float32),
        ]
    scratch += [pltpu.SemaphoreType.DMA] * (8 if DT else 4)

    @functools.partial(
        pl.kernel,
        mesh=mesh,
        out_type=jax.ShapeDtypeStruct((B, D), jnp.float32),
        scratch_types=scratch,
    )
    def gather_rows(table_hbm, idx_hbm, tail_hbm, out_hbm, idx_v,
                    buf0, buf1, *rest):
        if DT:
            tb0, tb1, gsem0, gsem1, tsem0, tsem1, ssem0, ssem1, s2em0, s2em1 = rest
            tbufs = (tb0, tb1)
            tsems = (tsem0, tsem1)
            s2ems = (s2em0, s2em1)
        else:
            gsem0, gsem1, ssem0, ssem1 = rest
        wid = lax.axis_index("s") * NC + lax.axis_index("c")
        base = wid * b_per_w
        pltpu.sync_copy(idx_hbm.at[pl.ds(base, b_per_w)], idx_v)

        bufs = (buf0, buf1)
        gsems = (gsem0, gsem1)
        ssems = (ssem0, ssem1)

        def gather(c):
            g = pltpu.async_copy(
                table_hbm.at[idx_v.at[pl.ds(c * CH, CH)]],
                bufs[c % 2], gsems[c % 2])
            if DT:
                t = pltpu.async_copy(
                    tail_hbm.at[pl.ds(base + c * CH, CH)],
                    tbufs[c % 2], tsems[c % 2])
                return (g, t)
            return (g,)

        def scatter(c):
            s = pltpu.async_copy(
                bufs[c % 2].at[:, pl.ds(0, DA)],
                out_hbm.at[pl.ds(base + c * CH, CH), pl.ds(0, DA)],
                ssems[c % 2])
            if DT:
                t = pltpu.async_copy(
                    tbufs[c % 2],
                    out_hbm.at[pl.ds(base + c * CH, CH), pl.ds(DA, DT)],
                    s2ems[c % 2])
                return (s, t)
            return (s,)

        def wait(cps):
            for cp in cps:
                cp.wait()

        gathers = {0: gather(0)}
        scatters = {}
        for c in range(n_ch):
            if c + 1 < n_ch:
                if c - 1 >= 0:
                    wait(scatters[c - 1])
                gathers[c + 1] = gather(c + 1)
            wait(gathers[c])
            scatters[c] = scatter(c)
        if n_ch >= 2:
            wait(scatters[n_ch - 2])
        wait(scatters[n_ch - 1])

    return gather_rows


def kernel(inputs_idx, params):
    B = inputs_idx.shape[0]
    N, D = params.shape
    DP = (D + 127) // 128 * 128
    DA = D // 128 * 128
    DT = D - DA

    sm_table = pl.pallas_call(
        _softmax_pad_body,
        out_shape=jax.ShapeDtypeStruct((N, DP), jnp.float32),
    )(params)

    idx32 = inputs_idx.astype(jnp.int32)

    info = plsc.get_sparse_core_info()
    NC, NS = info.num_cores, info.num_subcores

    if DT:
        TB = 1024
        idx3 = idx32.reshape(B // TB, 1, TB)
        tail = pl.pallas_call(
            functools.partial(_tail_body, n=N, c0=DA, c1=D),
            grid=(B // TB,),
            in_specs=[
                pl.BlockSpec((1, 1, TB), lambda i: (i, 0, 0)),
                pl.BlockSpec((N, DP), lambda i: (0, 0)),
            ],
            out_specs=pl.BlockSpec((TB, DT), lambda i: (i, 0)),
            out_shape=jax.ShapeDtypeStruct((B, DT), jnp.float32),
        )(idx3, sm_table)
    else:
        tail = jnp.zeros((B, 0), jnp.float32)

    gather_rows = _make_sc_gather(B, D, DP, DA, NC, NS, CH=32)
    return gather_rows(sm_table, idx32, tail)


# speedup vs baseline: 6.7316x; 5.1269x over previous
import functools

import jax
import jax.numpy as jnp
from jax import lax
from jax.experimental import pallas as pl


def _softmax_body(x_ref, o_ref):
    x = x_ref[...]
    m = jnp.max(x, axis=-1, keepdims=True)
    e = jnp.exp(x - m)
    s = jnp.sum(e, axis=-1, keepdims=True)
    o_ref[...] = e / s


def _route_body(idx_ref, table_ref, o_ref, *, n):
    idx = idx_ref[0, 0, :]
    blk = idx.shape[0]
    onehot = (lax.broadcasted_iota(jnp.int32, (n, blk), 0)
              == idx[None, :]).astype(jnp.float32)
    o_ref[...] = lax.dot_general(
        table_ref[...], onehot, (((0,), (0,)), ((), ())),
        preferred_element_type=jnp.float32)


def kernel(inputs_idx, params):
    B = inputs_idx.shape[0]
    N, D = params.shape

    sm_table = pl.pallas_call(
        _softmax_body,
        out_shape=jax.ShapeDtypeStruct((N, D), jnp.float32),
    )(params)

    idx32 = inputs_idx.astype(jnp.int32)
    BLK = 512
    idx3 = idx32.reshape(B // BLK, 1, BLK)

    out_t = pl.pallas_call(
        functools.partial(_route_body, n=N),
        grid=(B // BLK,),
        in_specs=[
            pl.BlockSpec((1, 1, BLK), lambda i: (i, 0, 0)),
            pl.BlockSpec((N, D), lambda i: (0, 0)),
        ],
        out_specs=pl.BlockSpec((D, BLK), lambda i: (0, i)),
        out_shape=jax.ShapeDtypeStruct((D, B), jnp.float32),
    )(idx3, sm_table)

    return out_t.T


# speedup vs baseline: 8.8676x; 1.3173x over previous
import functools

import jax
import jax.numpy as jnp
from jax import lax
from jax.experimental import pallas as pl


def _softmax_body(x_ref, o_ref):
    x = x_ref[...]
    m = jnp.max(x, axis=-1, keepdims=True)
    e = jnp.exp(x - m)
    s = jnp.sum(e, axis=-1, keepdims=True)
    o_ref[...] = e / s


def _route_body(idx_ref, table_ref, o_ref, *, n):
    idx = idx_ref[0, 0, :]
    blk = idx.shape[0]
    onehot = (lax.broadcasted_iota(jnp.int32, (n, blk), 0)
              == idx[None, :]).astype(jnp.float32)
    o_ref[...] = lax.dot_general(
        table_ref[...], onehot, (((0,), (0,)), ((), ())),
        preferred_element_type=jnp.float32)


def kernel(inputs_idx, params):
    B = inputs_idx.shape[0]
    N, D = params.shape

    sm_table = pl.pallas_call(
        _softmax_body,
        out_shape=jax.ShapeDtypeStruct((N, D), jnp.float32),
    )(params)

    idx32 = inputs_idx.astype(jnp.int32)
    BLK = 1024
    idx3 = idx32.reshape(B // BLK, 1, BLK)

    out_t = pl.pallas_call(
        functools.partial(_route_body, n=N),
        grid=(B // BLK,),
        in_specs=[
            pl.BlockSpec((1, 1, BLK), lambda i: (i, 0, 0)),
            pl.BlockSpec((N, D), lambda i: (0, 0)),
        ],
        out_specs=pl.BlockSpec((D, BLK), lambda i: (0, i)),
        out_shape=jax.ShapeDtypeStruct((D, B), jnp.float32),
    )(idx3, sm_table)

    return out_t.T


# speedup vs baseline: 9.0474x; 1.0203x over previous
import functools

import jax
import jax.numpy as jnp
from jax import lax
from jax.experimental import pallas as pl


def _softmax_body(x_ref, o_ref):
    x = x_ref[...]
    m = jnp.max(x, axis=-1, keepdims=True)
    e = jnp.exp(x - m)
    s = jnp.sum(e, axis=-1, keepdims=True)
    o_ref[...] = e / s


def _route_body(idx_ref, table_ref, o_ref, *, n):
    idx = idx_ref[0, 0, :]
    blk = idx.shape[0]
    onehot = (lax.broadcasted_iota(jnp.int32, (n, blk), 0)
              == idx[None, :]).astype(jnp.float32)
    o_ref[...] = lax.dot_general(
        table_ref[...], onehot, (((0,), (0,)), ((), ())),
        preferred_element_type=jnp.float32)


def kernel(inputs_idx, params):
    B = inputs_idx.shape[0]
    N, D = params.shape

    sm_table = pl.pallas_call(
        _softmax_body,
        out_shape=jax.ShapeDtypeStruct((N, D), jnp.float32),
    )(params)

    idx32 = inputs_idx.astype(jnp.int32)
    BLK = 2048
    idx3 = idx32.reshape(B // BLK, 1, BLK)

    out_t = pl.pallas_call(
        functools.partial(_route_body, n=N),
        grid=(B // BLK,),
        in_specs=[
            pl.BlockSpec((1, 1, BLK), lambda i: (i, 0, 0)),
            pl.BlockSpec((N, D), lambda i: (0, 0)),
        ],
        out_specs=pl.BlockSpec((D, BLK), lambda i: (0, i)),
        out_shape=jax.ShapeDtypeStruct((D, B), jnp.float32),
    )(idx3, sm_table)

    return out_t.T
